# baseline (speedup 1.0000x reference)
import functools

import jax
import jax.numpy as jnp
from jax import lax
from jax.experimental import pallas as pl
from jax.experimental.pallas import tpu as pltpu
from jax.experimental.pallas import tpu_sc as plsc

_NUM_CORES = 2
_NUM_SUBCORES = 16
_NW = _NUM_CORES * _NUM_SUBCORES
_CHUNK = 128
_LANES = 16


@functools.lru_cache(maxsize=None)
def _build(vocab, dim, batch):
    b_per_w = batch // _NW
    n_chunks = b_per_w // _CHUNK
    groups = b_per_w // _LANES
    mesh = plsc.VectorSubcoreMesh(core_axis_name="c", subcore_axis_name="s")

    @functools.partial(
        pl.kernel,
        out_type=jax.ShapeDtypeStruct((_NW, b_per_w), jnp.float32),
        mesh=mesh,
        compiler_params=pltpu.CompilerParams(needs_layout_passes=False, use_tc_tiling_on_sc=False),
        scratch_types=[
            pltpu.VMEM((n_chunks, _CHUNK), jnp.int32),
            pltpu.VMEM((n_chunks, _CHUNK), jnp.int32),
            pltpu.VMEM((b_per_w, dim), jnp.float32),
            pltpu.VMEM((b_per_w, dim), jnp.float32),
            pltpu.VMEM((b_per_w,), jnp.float32),
            pltpu.VMEM((b_per_w,), jnp.float32),
            pltpu.VMEM((b_per_w,), jnp.float32),
            pltpu.SemaphoreType.DMA,
        ],
    )
    def glove_kernel(w_hbm, wt_hbm, b_hbm, bt_hbm, i_hbm, j_hbm, out_hbm,
                     ii_v, jj_v, wi_v, wj_v, bi_v, bj_v, out_v, sem):
        wid = lax.axis_index("s") * _NUM_CORES + lax.axis_index("c")
        pltpu.sync_copy(i_hbm.at[wid], ii_v)
        pltpu.sync_copy(j_hbm.at[wid], jj_v)
        copies = []
        for k in range(n_chunks):
            sl = pl.ds(k * _CHUNK, _CHUNK)
            copies.append(pltpu.async_copy(w_hbm.at[ii_v.at[k]], wi_v.at[sl], sem))
            copies.append(pltpu.async_copy(wt_hbm.at[jj_v.at[k]], wj_v.at[sl], sem))
            copies.append(pltpu.async_copy(b_hbm.at[ii_v.at[k]], bi_v.at[sl], sem))
            copies.append(pltpu.async_copy(bt_hbm.at[jj_v.at[k]], bj_v.at[sl], sem))
        for cp in copies:
            cp.wait()

        def group_body(g, carry):
            rows = g * _LANES + lax.iota(jnp.int32, _LANES)
            acc = bi_v[pl.ds(g * _LANES, _LANES)] + bj_v[pl.ds(g * _LANES, _LANES)]
            for d in range(dim):
                cols = jnp.full((_LANES,), d, jnp.int32)
                acc = acc + (plsc.load_gather(wi_v, [rows, cols])
                             * plsc.load_gather(wj_v, [rows, cols]))
            out_v[pl.ds(g * _LANES, _LANES)] = acc
            return carry

        lax.fori_loop(0, groups, group_body, 0)
        pltpu.sync_copy(out_v, out_hbm.at[wid])

    return glove_kernel


def kernel(W, W_tilde, b, b_tilde, i_idx, j_idx):
    vocab, dim = W.shape
    batch = i_idx.shape[0]
    b_per_w = batch // _NW
    fn = _build(vocab, dim, batch)
    out = fn(
        W,
        W_tilde,
        b.reshape(vocab),
        b_tilde.reshape(vocab),
        i_idx.reshape(_NW, b_per_w // _CHUNK, _CHUNK),
        j_idx.reshape(_NW, b_per_w // _CHUNK, _CHUNK),
    )
    return out.reshape(batch)
